# baseline (speedup 1.0000x reference)
import functools

import jax
import jax.numpy as jnp
from jax import lax
from jax.experimental import pallas as pl
from jax.experimental.pallas import tpu as pltpu
from jax.experimental.pallas import tpu_sc as plsc

_D = 768
_E = 64
_N = 32768
_BLK = 4096
_NC = 2
_NS = 16
_NW = _NC * _NS
_TPW = _N // _NW
_L = 16
_G = 4


def _matmul_body(x_ref, wt_ref, b_ref, k_ref):
    x = x_ref[...]
    wt = wt_ref[...]
    s = jnp.dot(x, wt, preferred_element_type=jnp.float32)
    s = s + b_ref[...]
    bits = lax.bitcast_convert_type(s, jnp.int32)
    key = jnp.where(bits >= 0, bits, bits ^ 0x7FFFFFFF)
    iota_e = lax.broadcasted_iota(jnp.int32, s.shape, 1)
    k_ref[...] = (key & ~0x3F) | (63 - iota_e)


def _tc_keys(inputs, wt, brow):
    return pl.pallas_call(
        _matmul_body,
        grid=(_N // _BLK,),
        in_specs=[
            pl.BlockSpec((_BLK, _D), lambda i: (i, 0)),
            pl.BlockSpec((_D, _E), lambda i: (0, 0)),
            pl.BlockSpec((1, _E), lambda i: (0, 0)),
        ],
        out_specs=pl.BlockSpec((_BLK, _E), lambda i: (i, 0)),
        out_shape=jax.ShapeDtypeStruct((_N, _E), jnp.int32),
        compiler_params=pltpu.CompilerParams(
            dimension_semantics=("arbitrary",),
        ),
    )(inputs, wt, brow)


def _decode(k):
    e = 63 - (k & 0x3F)
    kf = k | 0x20
    bits = jnp.where(kf >= 0, kf, kf ^ 0x7FFFFFFF)
    s = lax.bitcast_convert_type(bits, jnp.float32)
    return s, e


def _sc_body(k_hbm, p_hbm, i_hbm, kbuf, pbuf, ibuf):
    wid = lax.axis_index("s") * _NC + lax.axis_index("c")
    base = wid * _TPW
    pltpu.sync_copy(k_hbm.at[pl.ds(base * _E, _TPW * _E)], kbuf)

    lanes = lax.iota(jnp.int32, _L)
    minint = jnp.full((_L,), -0x80000000, jnp.int32)

    def block(blk, carry):
        flats = [(blk * (_G * _L) + c * _L + lanes) * _E for c in range(_G)]
        m1 = [plsc.load_gather(kbuf, [f]) for f in flats]
        m2 = [minint] * _G
        for e in range(1, _E):
            v = [plsc.load_gather(kbuf, [f + e]) for f in flats]
            for c in range(_G):
                m2[c] = jnp.maximum(m2[c], jnp.minimum(v[c], m1[c]))
                m1[c] = jnp.maximum(m1[c], v[c])
        for c in range(_G):
            s1, e1 = _decode(m1[c])
            s2, e2 = _decode(m2[c])
            x2 = jnp.exp(s2 - s1)
            p1 = 1.0 / (1.0 + x2)
            p2 = 1.0 - p1
            out = (blk * (_G * _L) + c * _L + lanes) * 2
            plsc.store_scatter(pbuf, [out], p1)
            plsc.store_scatter(pbuf, [out + 1], p2)
            plsc.store_scatter(ibuf, [out], e1)
            plsc.store_scatter(ibuf, [out + 1], e2)
        return carry

    lax.fori_loop(0, _TPW // (_G * _L), block, 0)
    pltpu.sync_copy(pbuf, p_hbm.at[pl.ds(base * 2, _TPW * 2)])
    pltpu.sync_copy(ibuf, i_hbm.at[pl.ds(base * 2, _TPW * 2)])


def _sc_topk(keys):
    mesh = plsc.VectorSubcoreMesh(
        core_axis_name="c", subcore_axis_name="s",
        num_cores=_NC, num_subcores=_NS)
    return pl.kernel(
        _sc_body,
        out_type=[
            jax.ShapeDtypeStruct((_N * 2,), jnp.float32),
            jax.ShapeDtypeStruct((_N * 2,), jnp.int32),
        ],
        mesh=mesh,
        compiler_params=pltpu.CompilerParams(needs_layout_passes=False),
        scratch_types=[
            pltpu.VMEM((_TPW * _E,), jnp.int32),
            pltpu.VMEM((_TPW * 2,), jnp.float32),
            pltpu.VMEM((_TPW * 2,), jnp.int32),
        ],
    )(keys.reshape(-1))


def kernel(inputs, W, b):
    wt = W.T
    brow = b.reshape(1, _E)
    keys = _tc_keys(inputs, wt, brow)
    probs, idx = _sc_topk(keys)
    return (probs.reshape(_N, 2), idx.reshape(_N, 2))


# speedup vs baseline: 1.0776x; 1.0776x over previous
import functools

import jax
import jax.numpy as jnp
from jax import lax
from jax.experimental import pallas as pl
from jax.experimental.pallas import tpu as pltpu
from jax.experimental.pallas import tpu_sc as plsc

_D = 768
_E = 64
_N = 32768
_BLK = 4096
_NC = 2
_NS = 16
_NW = _NC * _NS
_TPW = _N // _NW
_L = 16
_G = 4


def _matmul_body(x_ref, wt_ref, b_ref, k_ref):
    x = x_ref[...]
    wt = wt_ref[...]
    s = jnp.dot(x, wt, preferred_element_type=jnp.float32)
    s = s + b_ref[...]
    bits = lax.bitcast_convert_type(s, jnp.int32)
    key = jnp.where(bits >= 0, bits, bits ^ 0x7FFFFFFF)
    iota_e = lax.broadcasted_iota(jnp.int32, s.shape, 1)
    k_ref[...] = (key & ~0x3F) | (63 - iota_e)


def _tc_keys(inputs, wt, brow):
    return pl.pallas_call(
        _matmul_body,
        grid=(_N // _BLK,),
        in_specs=[
            pl.BlockSpec((_BLK, _D), lambda i: (i, 0)),
            pl.BlockSpec((_D, _E), lambda i: (0, 0)),
            pl.BlockSpec((1, _E), lambda i: (0, 0)),
        ],
        out_specs=pl.BlockSpec((_BLK, _E), lambda i: (i, 0)),
        out_shape=jax.ShapeDtypeStruct((_N, _E), jnp.int32),
        compiler_params=pltpu.CompilerParams(
            dimension_semantics=("arbitrary",),
        ),
    )(inputs, wt, brow)


def _decode(k):
    e = 63 - (k & 0x3F)
    kf = k | 0x20
    bits = jnp.where(kf >= 0, kf, kf ^ 0x7FFFFFFF)
    s = lax.bitcast_convert_type(bits, jnp.float32)
    return s, e


_CH = 256
_NCH = _TPW // _CH


def _sc_body(k_hbm, p_hbm, i_hbm, kbuf, pbuf, ibuf, sem0, sem1):
    wid = lax.axis_index("s") * _NC + lax.axis_index("c")
    base = wid * _TPW
    sems = (sem0, sem1)

    lanes = lax.iota(jnp.int32, _L)
    minint = jnp.full((_L,), -0x80000000, jnp.int32)
    zcol = jnp.zeros((_L,), jnp.int32)
    ocol = zcol + 1

    def start(ch, nb):
        pltpu.async_copy(
            k_hbm.at[pl.ds(base + ch * _CH, _CH)], kbuf.at[nb], sems[nb])

    def wait(ch, nb):
        pltpu.make_async_copy(
            k_hbm.at[pl.ds(base + ch * _CH, _CH)], kbuf.at[nb],
            sems[nb]).wait()

    def compute(ch, nb):
        kb = kbuf.at[nb]

        def blk_body(blk, carry):
            loc = [blk * (_G * _L) + c * _L + lanes for c in range(_G)]
            m1 = [plsc.load_gather(kb, [r, zcol]) for r in loc]
            m2 = [minint] * _G
            for e in range(1, _E):
                col = zcol + e
                v = [plsc.load_gather(kb, [r, col]) for r in loc]
                for c in range(_G):
                    m2[c] = jnp.maximum(m2[c], jnp.minimum(v[c], m1[c]))
                    m1[c] = jnp.maximum(m1[c], v[c])
            for c in range(_G):
                s1, e1 = _decode(m1[c])
                s2, e2 = _decode(m2[c])
                x2 = jnp.exp(s2 - s1)
                p1 = 1.0 / (1.0 + x2)
                p2 = 1.0 - p1
                out = (ch * _CH + loc[c]) * 2
                plsc.store_scatter(pbuf, [out], p1)
                plsc.store_scatter(pbuf, [out + 1], p2)
                plsc.store_scatter(ibuf, [out], e1)
                plsc.store_scatter(ibuf, [out + 1], e2)
            return carry

        lax.fori_loop(0, _CH // (_G * _L), blk_body, 0)

    start(0, 0)

    def chunk(ch, carry):
        def phase(nb, other):
            @pl.when(ch + 1 < _NCH)
            def _():
                start(ch + 1, other)
            wait(ch, nb)
            compute(ch, nb)
            return 0

        lax.cond(ch % 2 == 0, lambda: phase(0, 1), lambda: phase(1, 0))
        return carry

    lax.fori_loop(0, _NCH, chunk, 0)

    pltpu.sync_copy(pbuf, p_hbm.at[pl.ds(base * 2, _TPW * 2)])
    pltpu.sync_copy(ibuf, i_hbm.at[pl.ds(base * 2, _TPW * 2)])


def _sc_topk(keys):
    mesh = plsc.VectorSubcoreMesh(
        core_axis_name="c", subcore_axis_name="s",
        num_cores=_NC, num_subcores=_NS)
    return pl.kernel(
        _sc_body,
        out_type=[
            jax.ShapeDtypeStruct((_N * 2,), jnp.float32),
            jax.ShapeDtypeStruct((_N * 2,), jnp.int32),
        ],
        mesh=mesh,
        compiler_params=pltpu.CompilerParams(needs_layout_passes=False),
        scratch_types=[
            pltpu.VMEM((2, _CH, _E), jnp.int32),
            pltpu.VMEM((_TPW * 2,), jnp.float32),
            pltpu.VMEM((_TPW * 2,), jnp.int32),
            pltpu.SemaphoreType.DMA,
            pltpu.SemaphoreType.DMA,
        ],
    )(keys)


def kernel(inputs, W, b):
    wt = W.T
    brow = b.reshape(1, _E)
    keys = _tc_keys(inputs, wt, brow)
    probs, idx = _sc_topk(keys)
    return (probs.reshape(_N, 2), idx.reshape(_N, 2))


# speedup vs baseline: 2.2980x; 2.1326x over previous
import jax
import jax.numpy as jnp
from jax import lax
from jax.experimental import pallas as pl
from jax.experimental.pallas import tpu as pltpu

_D = 768
_E = 64
_N = 32768
_BLK = 4096


def _body(x_ref, wt_ref, b_ref, p_ref, i_ref):
    x = x_ref[...]
    wt = wt_ref[...]
    s = jnp.dot(x, wt, preferred_element_type=jnp.float32)
    s = s + b_ref[...]
    bits = lax.bitcast_convert_type(s, jnp.int32)
    key = jnp.where(bits >= 0, bits, bits ^ 0x7FFFFFFF)
    iota_e = lax.broadcasted_iota(jnp.int32, s.shape, 1)
    key = (key & ~0x3F) | (63 - iota_e)

    m1 = jnp.max(key, axis=1, keepdims=True)
    masked = jnp.where(key == m1, jnp.int32(-0x80000000), key)
    m2 = jnp.max(masked, axis=1, keepdims=True)

    def dec(k):
        e = 63 - (k & 0x3F)
        kf = k | 0x20
        b2 = jnp.where(kf >= 0, kf, kf ^ 0x7FFFFFFF)
        return lax.bitcast_convert_type(b2, jnp.float32), e

    s1, e1 = dec(m1)
    s2, e2 = dec(m2)
    x2 = jnp.exp(s2 - s1)
    p1 = 1.0 / (1.0 + x2)
    p2 = 1.0 - p1
    p_ref[...] = jnp.concatenate([p1, p2], axis=1)
    i_ref[...] = jnp.concatenate([e1, e2], axis=1)


def kernel(inputs, W, b):
    wt = W.T
    brow = b.reshape(1, _E)
    probs, idx = pl.pallas_call(
        _body,
        grid=(_N // _BLK,),
        in_specs=[
            pl.BlockSpec((_BLK, _D), lambda i: (i, 0)),
            pl.BlockSpec((_D, _E), lambda i: (0, 0)),
            pl.BlockSpec((1, _E), lambda i: (0, 0)),
        ],
        out_specs=[
            pl.BlockSpec((_BLK, 2), lambda i: (i, 0)),
            pl.BlockSpec((_BLK, 2), lambda i: (i, 0)),
        ],
        out_shape=[
            jax.ShapeDtypeStruct((_N, 2), jnp.float32),
            jax.ShapeDtypeStruct((_N, 2), jnp.int32),
        ],
        compiler_params=pltpu.CompilerParams(
            dimension_semantics=("arbitrary",),
        ),
    )(inputs, wt, brow)
    return (probs, idx)
